# baseline (speedup 1.0000x reference)
import dataclasses
import functools

import jax
import jax.numpy as jnp
from jax import lax
from jax.experimental import pallas as pl
from jax.experimental.pallas import tpu as pltpu
from jax.experimental.pallas import tpu_sc as plsc

N = 10000
D = 128
E = 320000
K = 128
G = 8
W_GATHER = 2
E_PAD = 327680
NCHT = E_PAD // K
WCH = NCHT // 32
WG = WCH // G
RPS = 624
RTAIL = N - 16 * RPS
N_ACC = 10240
BN = 1000
NB = N // BN


def _mesh():
    return plsc.VectorSubcoreMesh(core_axis_name="c", subcore_axis_name="s")


EW = E_PAD // 32
N_H = N_ACC


def _sc_hist(colsf):
    @functools.partial(
        pl.kernel,
        out_type=jax.ShapeDtypeStruct((32, 1, N_H), jnp.float32),
        mesh=_mesh(),
        scratch_types=[
            pltpu.VMEM((EW,), jnp.int32),
            pltpu.VMEM((1, N_H), jnp.float32),
            pltpu.SemaphoreType.DMA,
        ],
        compiler_params=dataclasses.replace(pltpu.CompilerParams(),
                                            needs_layout_passes=False),
    )
    def k(c_hbm, out_hbm, colb, hv, isem):
        cid = lax.axis_index("c")
        sid = lax.axis_index("s")
        wid = sid * 2 + cid
        cp = pltpu.async_copy(c_hbm.at[pl.ds(wid * EW, EW)], colb, isem)

        z16 = jnp.zeros((16,), jnp.float32)

        @pl.loop(0, N_H // 16)
        def _(i):
            hv[0, pl.ds(i * 16, 16)] = z16

        cp.wait()
        zi = jnp.zeros((16,), jnp.int32)
        o16 = jnp.ones((16,), jnp.float32)

        @pl.loop(0, EW // 16)
        def _(i):
            idx = colb[pl.ds(i * 16, 16)]
            plsc.addupdate_scatter(hv, [zi, idx], o16)

        pltpu.sync_copy(hv, out_hbm.at[wid])

    return k(colsf)


def _dis(hist32):
    bh = 2560

    def body(h_ref, o_ref):
        s = jnp.sum(h_ref[...], axis=0, keepdims=True) + 1.0
        o_ref[...] = lax.rsqrt(s).reshape(bh, 1)

    return pl.pallas_call(
        body,
        grid=(N_H // bh,),
        in_specs=[pl.BlockSpec((32, bh), lambda i: (0, i))],
        out_specs=pl.BlockSpec((bh, 1), lambda i: (i, 0)),
        out_shape=jax.ShapeDtypeStruct((N_H, 1), jnp.float32),
    )(hist32)


ZR = 48


def _sc_agg(y, rows2, cols2):
    @functools.partial(
        pl.kernel,
        out_type=jax.ShapeDtypeStruct((2 * N, D), jnp.float32),
        mesh=_mesh(),
        scratch_types=[
            pltpu.VMEM((G, K), jnp.int32),
            pltpu.VMEM((G, K), jnp.int32),
            pltpu.VMEM((G, K), jnp.int32),
            pltpu.VMEM((G, K), jnp.int32),
            pltpu.VMEM((W_GATHER, K, D), jnp.float32),
            pltpu.VMEM((ZR, D), jnp.float32),
            pltpu.VMEM_SHARED((N_ACC, D), jnp.float32),
            pltpu.SemaphoreType.DMA,
            pltpu.SemaphoreType.DMA,
            pltpu.SemaphoreType.DMA,
            pltpu.SemaphoreType.DMA,
            pltpu.SemaphoreType.DMA,
            pltpu.SemaphoreType.DMA,
            pltpu.SemaphoreType.DMA,
        ],
    )
    def k(y_hbm, r_hbm, c_hbm, out_hbm, rowb0, colb0, rowb1, colb1, gbuf,
          zbuf, acc, gsem0, gsem1, ssem0, ssem1, zsem, isem0, isem1):
        cid = lax.axis_index("c")
        sid = lax.axis_index("s")
        wid = sid * 2 + cid
        rbase = sid * RPS

        z16 = jnp.zeros((16,), jnp.float32)

        @pl.loop(0, ZR)
        def _(r):
            @pl.loop(0, D // 16)
            def _(c):
                zbuf[r, pl.ds(c * 16, 16)] = z16

        zcp = [pltpu.async_copy(zbuf, acc.at[pl.ds(rbase + i * ZR, ZR)], zsem)
               for i in range(RPS // ZR)]

        @pl.when(sid == 0)
        def _():
            pltpu.async_copy(zbuf.at[pl.ds(0, RTAIL)],
                             acc.at[pl.ds(16 * RPS, RTAIL)], zsem).wait()

        cbase = wid * WCH
        gsems = (gsem0, gsem1)
        ssems = (ssem0, ssem1)
        rowbs = (rowb0, rowb1)
        colbs = (colb0, colb1)
        isems = (isem0, isem1)

        def load_idx(slot, gidx):
            ch = cbase + gidx * G
            pltpu.async_copy(r_hbm.at[pl.ds(ch, G)], rowbs[slot], isems[slot])
            pltpu.async_copy(c_hbm.at[pl.ds(ch, G)], colbs[slot], isems[slot])

        def wait_idx(slot):
            pltpu.make_async_copy(r_hbm.at[pl.ds(cbase, G)], rowbs[slot],
                                  isems[slot]).wait()
            pltpu.make_async_copy(c_hbm.at[pl.ds(cbase, G)], colbs[slot],
                                  isems[slot]).wait()

        def process(slot):
            rowb = rowbs[slot]
            colb = colbs[slot]
            gcp = [None] * G
            scp = [None] * G
            gcp[0] = pltpu.async_copy(y_hbm.at[rowb.at[0]], gbuf.at[0], gsems[0])
            gcp[1] = pltpu.async_copy(y_hbm.at[rowb.at[1]], gbuf.at[1], gsems[1])
            for j in range(G):
                b = j % 2
                if 1 <= j < G - 1:
                    scp[j - 1].wait()
                    gcp[j + 1] = pltpu.async_copy(y_hbm.at[rowb.at[j + 1]],
                                                  gbuf.at[1 - b], gsems[1 - b])
                gcp[j].wait()
                scp[j] = pltpu.async_copy(gbuf.at[b], acc.at[colb.at[j]],
                                          ssems[b], add=True)
            scp[G - 2].wait()
            scp[G - 1].wait()

        load_idx(0, 0)
        for cp in zcp:
            cp.wait()
        plsc.subcore_barrier()

        @pl.loop(0, WG, step=2)
        def _(g):
            load_idx(1, g + 1)
            wait_idx(0)
            process(0)
            wait_idx(1)

            @pl.when(g + 2 < WG)
            def _():
                load_idx(0, g + 2)

            process(1)

        plsc.subcore_barrier()
        pltpu.sync_copy(acc.at[pl.ds(rbase, RPS)],
                        out_hbm.at[pl.ds(cid * N + rbase, RPS)])

        @pl.when(sid == 0)
        def _():
            pltpu.sync_copy(acc.at[pl.ds(16 * RPS, RTAIL)],
                            out_hbm.at[pl.ds(cid * N + 16 * RPS, RTAIL)])

    return k(y, rows2, cols2)


def _mm_first(x, w, dis):
    def body(x_ref, w_ref, d_ref, o_ref):
        o_ref[...] = jnp.dot(x_ref[...], w_ref[...],
                             precision=lax.Precision.HIGHEST) * d_ref[...]

    return pl.pallas_call(
        body,
        grid=(NB,),
        in_specs=[
            pl.BlockSpec((BN, D), lambda i: (i, 0)),
            pl.BlockSpec((D, D), lambda i: (0, 0)),
            pl.BlockSpec((BN, 1), lambda i: (i, 0)),
        ],
        out_specs=pl.BlockSpec((BN, D), lambda i: (i, 0)),
        out_shape=jax.ShapeDtypeStruct((N, D), jnp.float32),
    )(x, w, dis)


def _mm_mid(a, y, dis, b1, w2):
    def body(p0_ref, p1_ref, y_ref, d_ref, b_ref, w_ref, o_ref):
        dis = d_ref[...]
        full = p0_ref[...] + p1_ref[...] + y_ref[...]
        hid = jnp.maximum(full * dis + b_ref[...], 0.0)
        o_ref[...] = jnp.dot(hid, w_ref[...],
                             precision=lax.Precision.HIGHEST) * dis

    return pl.pallas_call(
        body,
        grid=(NB,),
        in_specs=[
            pl.BlockSpec((BN, D), lambda i: (i, 0)),
            pl.BlockSpec((BN, D), lambda i: (i + NB, 0)),
            pl.BlockSpec((BN, D), lambda i: (i, 0)),
            pl.BlockSpec((BN, 1), lambda i: (i, 0)),
            pl.BlockSpec((1, D), lambda i: (0, 0)),
            pl.BlockSpec((D, D), lambda i: (0, 0)),
        ],
        out_specs=pl.BlockSpec((BN, D), lambda i: (i, 0)),
        out_shape=jax.ShapeDtypeStruct((N, D), jnp.float32),
    )(a, a, y, dis, b1, w2)


def _mm_final(a, y, dis, b2, wh, bh):
    def body(p0_ref, p1_ref, y_ref, d_ref, b_ref, w_ref, bh_ref, o_ref):
        dis = d_ref[...]
        full = p0_ref[...] + p1_ref[...] + y_ref[...]
        z = full * dis + b_ref[...]
        o_ref[...] = jnp.dot(z, w_ref[...],
                             precision=lax.Precision.HIGHEST) + bh_ref[...]

    return pl.pallas_call(
        body,
        grid=(NB,),
        in_specs=[
            pl.BlockSpec((BN, D), lambda i: (i, 0)),
            pl.BlockSpec((BN, D), lambda i: (i + NB, 0)),
            pl.BlockSpec((BN, D), lambda i: (i, 0)),
            pl.BlockSpec((BN, 1), lambda i: (i, 0)),
            pl.BlockSpec((1, D), lambda i: (0, 0)),
            pl.BlockSpec((D, D), lambda i: (0, 0)),
            pl.BlockSpec((1, D), lambda i: (0, 0)),
        ],
        out_specs=pl.BlockSpec((BN, D), lambda i: (i, 0)),
        out_shape=jax.ShapeDtypeStruct((N, D), jnp.float32),
    )(a, a, y, dis, b2, wh, bh)


def kernel(x, edge_index, W1, b1, W2, b2, Wh, bh):
    row = edge_index[0]
    col = edge_index[1]
    pad = E_PAD - E
    pad_iota = jnp.arange(pad, dtype=jnp.int32)
    rows2 = jnp.concatenate([row, pad_iota % N]).reshape(NCHT, K)
    cols_p = jnp.concatenate([col, N + pad_iota % (N_ACC - N)])
    cols2 = cols_p.reshape(NCHT, K)
    b1r = b1.reshape(1, D)
    b2r = b2.reshape(1, D)
    bhr = bh.reshape(1, D)

    hist32 = _sc_hist(cols_p).reshape(32, N_H)
    dis = _dis(hist32)
    y1 = _mm_first(x, W1, dis)
    a1 = _sc_agg(y1, rows2, cols2)
    y2 = _mm_mid(a1, y1, dis, b1r, W2)
    a2 = _sc_agg(y2, rows2, cols2)
    return _mm_final(a2, y2, dis, b2r, Wh, bhr)


# speedup vs baseline: 1.0044x; 1.0044x over previous
import dataclasses
import functools

import jax
import jax.numpy as jnp
from jax import lax
from jax.experimental import pallas as pl
from jax.experimental.pallas import tpu as pltpu
from jax.experimental.pallas import tpu_sc as plsc

N = 10000
D = 128
E = 320000
K = 128
G = 16
W_GATHER = 2
E_PAD = 327680
NCHT = E_PAD // K
WCH = NCHT // 32
WG = WCH // G
RPS = 624
RTAIL = N - 16 * RPS
N_ACC = 10240
BN = 1000
NB = N // BN


def _mesh():
    return plsc.VectorSubcoreMesh(core_axis_name="c", subcore_axis_name="s")


EW = E_PAD // 32
N_H = N_ACC


def _sc_hist(colsf):
    @functools.partial(
        pl.kernel,
        out_type=jax.ShapeDtypeStruct((32, 1, N_H), jnp.float32),
        mesh=_mesh(),
        scratch_types=[
            pltpu.VMEM((EW,), jnp.int32),
            pltpu.VMEM((1, N_H), jnp.float32),
            pltpu.SemaphoreType.DMA,
        ],
        compiler_params=dataclasses.replace(pltpu.CompilerParams(),
                                            needs_layout_passes=False),
    )
    def k(c_hbm, out_hbm, colb, hv, isem):
        cid = lax.axis_index("c")
        sid = lax.axis_index("s")
        wid = sid * 2 + cid
        cp = pltpu.async_copy(c_hbm.at[pl.ds(wid * EW, EW)], colb, isem)

        z16 = jnp.zeros((16,), jnp.float32)

        @pl.loop(0, N_H // 16)
        def _(i):
            hv[0, pl.ds(i * 16, 16)] = z16

        cp.wait()
        zi = jnp.zeros((16,), jnp.int32)
        o16 = jnp.ones((16,), jnp.float32)

        @pl.loop(0, EW // 16)
        def _(i):
            idx = colb[pl.ds(i * 16, 16)]
            plsc.addupdate_scatter(hv, [zi, idx], o16)

        pltpu.sync_copy(hv, out_hbm.at[wid])

    return k(colsf)


def _dis(hist32):
    bh = 2560

    def body(h_ref, o_ref):
        s = jnp.sum(h_ref[...], axis=0, keepdims=True) + 1.0
        o_ref[...] = lax.rsqrt(s).reshape(bh, 1)

    return pl.pallas_call(
        body,
        grid=(N_H // bh,),
        in_specs=[pl.BlockSpec((32, bh), lambda i: (0, i))],
        out_specs=pl.BlockSpec((bh, 1), lambda i: (i, 0)),
        out_shape=jax.ShapeDtypeStruct((N_H, 1), jnp.float32),
    )(hist32)


ZR = 48


def _sc_agg(y, rows2, cols2):
    @functools.partial(
        pl.kernel,
        out_type=jax.ShapeDtypeStruct((2 * N, D), jnp.float32),
        mesh=_mesh(),
        scratch_types=[
            pltpu.VMEM((G, K), jnp.int32),
            pltpu.VMEM((G, K), jnp.int32),
            pltpu.VMEM((W_GATHER, K, D), jnp.float32),
            pltpu.VMEM((ZR, D), jnp.float32),
            pltpu.VMEM_SHARED((N_ACC, D), jnp.float32),
            pltpu.SemaphoreType.DMA,
            pltpu.SemaphoreType.DMA,
            pltpu.SemaphoreType.DMA,
            pltpu.SemaphoreType.DMA,
            pltpu.SemaphoreType.DMA,
        ],
    )
    def k(y_hbm, r_hbm, c_hbm, out_hbm, rowb0, colb0, gbuf,
          zbuf, acc, gsem0, gsem1, ssem0, ssem1, zsem):
        cid = lax.axis_index("c")
        sid = lax.axis_index("s")
        wid = sid * 2 + cid
        rbase = sid * RPS

        z16 = jnp.zeros((16,), jnp.float32)

        @pl.loop(0, ZR)
        def _(r):
            @pl.loop(0, D // 16)
            def _(c):
                zbuf[r, pl.ds(c * 16, 16)] = z16

        zcp = [pltpu.async_copy(zbuf, acc.at[pl.ds(rbase + i * ZR, ZR)], zsem)
               for i in range(RPS // ZR)]

        @pl.when(sid == 0)
        def _():
            pltpu.async_copy(zbuf.at[pl.ds(0, RTAIL)],
                             acc.at[pl.ds(16 * RPS, RTAIL)], zsem).wait()

        cbase = wid * WCH
        gsems = (gsem0, gsem1)
        ssems = (ssem0, ssem1)

        def process(rowb, colb):
            gcp = [None] * G
            scp = [None] * G
            gcp[0] = pltpu.async_copy(y_hbm.at[rowb.at[0]], gbuf.at[0], gsems[0])
            gcp[1] = pltpu.async_copy(y_hbm.at[rowb.at[1]], gbuf.at[1], gsems[1])
            for j in range(G):
                b = j % 2
                if 1 <= j < G - 1:
                    scp[j - 1].wait()
                    gcp[j + 1] = pltpu.async_copy(y_hbm.at[rowb.at[j + 1]],
                                                  gbuf.at[1 - b], gsems[1 - b])
                gcp[j].wait()
                scp[j] = pltpu.async_copy(gbuf.at[b], acc.at[colb.at[j]],
                                          ssems[b], add=True)
            scp[G - 2].wait()
            scp[G - 1].wait()

        for cp in zcp:
            cp.wait()
        plsc.subcore_barrier()

        @pl.loop(0, WG)
        def _(g):
            ch = cbase + g * G
            pltpu.sync_copy(r_hbm.at[pl.ds(ch, G)], rowb0)
            pltpu.sync_copy(c_hbm.at[pl.ds(ch, G)], colb0)
            process(rowb0, colb0)

        plsc.subcore_barrier()
        pltpu.sync_copy(acc.at[pl.ds(rbase, RPS)],
                        out_hbm.at[pl.ds(cid * N + rbase, RPS)])

        @pl.when(sid == 0)
        def _():
            pltpu.sync_copy(acc.at[pl.ds(16 * RPS, RTAIL)],
                            out_hbm.at[pl.ds(cid * N + 16 * RPS, RTAIL)])

    return k(y, rows2, cols2)


def _mm_first(x, w, dis):
    def body(x_ref, w_ref, d_ref, o_ref):
        o_ref[...] = jnp.dot(x_ref[...], w_ref[...],
                             precision=lax.Precision.HIGHEST) * d_ref[...]

    return pl.pallas_call(
        body,
        grid=(NB,),
        in_specs=[
            pl.BlockSpec((BN, D), lambda i: (i, 0)),
            pl.BlockSpec((D, D), lambda i: (0, 0)),
            pl.BlockSpec((BN, 1), lambda i: (i, 0)),
        ],
        out_specs=pl.BlockSpec((BN, D), lambda i: (i, 0)),
        out_shape=jax.ShapeDtypeStruct((N, D), jnp.float32),
    )(x, w, dis)


def _mm_mid(a, y, dis, b1, w2):
    def body(p0_ref, p1_ref, y_ref, d_ref, b_ref, w_ref, o_ref):
        dis = d_ref[...]
        full = p0_ref[...] + p1_ref[...] + y_ref[...]
        hid = jnp.maximum(full * dis + b_ref[...], 0.0)
        o_ref[...] = jnp.dot(hid, w_ref[...],
                             precision=lax.Precision.HIGHEST) * dis

    return pl.pallas_call(
        body,
        grid=(NB,),
        in_specs=[
            pl.BlockSpec((BN, D), lambda i: (i, 0)),
            pl.BlockSpec((BN, D), lambda i: (i + NB, 0)),
            pl.BlockSpec((BN, D), lambda i: (i, 0)),
            pl.BlockSpec((BN, 1), lambda i: (i, 0)),
            pl.BlockSpec((1, D), lambda i: (0, 0)),
            pl.BlockSpec((D, D), lambda i: (0, 0)),
        ],
        out_specs=pl.BlockSpec((BN, D), lambda i: (i, 0)),
        out_shape=jax.ShapeDtypeStruct((N, D), jnp.float32),
    )(a, a, y, dis, b1, w2)


def _mm_final(a, y, dis, b2, wh, bh):
    def body(p0_ref, p1_ref, y_ref, d_ref, b_ref, w_ref, bh_ref, o_ref):
        dis = d_ref[...]
        full = p0_ref[...] + p1_ref[...] + y_ref[...]
        z = full * dis + b_ref[...]
        o_ref[...] = jnp.dot(z, w_ref[...],
                             precision=lax.Precision.HIGHEST) + bh_ref[...]

    return pl.pallas_call(
        body,
        grid=(NB,),
        in_specs=[
            pl.BlockSpec((BN, D), lambda i: (i, 0)),
            pl.BlockSpec((BN, D), lambda i: (i + NB, 0)),
            pl.BlockSpec((BN, D), lambda i: (i, 0)),
            pl.BlockSpec((BN, 1), lambda i: (i, 0)),
            pl.BlockSpec((1, D), lambda i: (0, 0)),
            pl.BlockSpec((D, D), lambda i: (0, 0)),
            pl.BlockSpec((1, D), lambda i: (0, 0)),
        ],
        out_specs=pl.BlockSpec((BN, D), lambda i: (i, 0)),
        out_shape=jax.ShapeDtypeStruct((N, D), jnp.float32),
    )(a, a, y, dis, b2, wh, bh)


def kernel(x, edge_index, W1, b1, W2, b2, Wh, bh):
    row = edge_index[0]
    col = edge_index[1]
    pad = E_PAD - E
    pad_iota = jnp.arange(pad, dtype=jnp.int32)
    rows2 = jnp.concatenate([row, pad_iota % N]).reshape(NCHT, K)
    cols_p = jnp.concatenate([col, N + pad_iota % (N_ACC - N)])
    cols2 = cols_p.reshape(NCHT, K)
    b1r = b1.reshape(1, D)
    b2r = b2.reshape(1, D)
    bhr = bh.reshape(1, D)

    hist32 = _sc_hist(cols_p).reshape(32, N_H)
    dis = _dis(hist32)
    y1 = _mm_first(x, W1, dis)
    a1 = _sc_agg(y1, rows2, cols2)
    y2 = _mm_mid(a1, y1, dis, b1r, W2)
    a2 = _sc_agg(y2, rows2, cols2)
    return _mm_final(a2, y2, dis, b2r, Wh, bhr)


# speedup vs baseline: 1.0416x; 1.0370x over previous
import dataclasses
import functools

import jax
import jax.numpy as jnp
from jax import lax
from jax.experimental import pallas as pl
from jax.experimental.pallas import tpu as pltpu
from jax.experimental.pallas import tpu_sc as plsc

N = 10000
D = 128
E = 320000
K = 128
G = 16
W_GATHER = 2
E_PAD = 327680
NCHT = E_PAD // K
WCH = NCHT // 32
WG = WCH // G
RPS = 624
RTAIL = N - 16 * RPS
N_ACC = 10240
BN = 2000
NB = N // BN


def _mesh():
    return plsc.VectorSubcoreMesh(core_axis_name="c", subcore_axis_name="s")


EW = E_PAD // 32
N_H = N_ACC


def _sc_hist(colsf):
    @functools.partial(
        pl.kernel,
        out_type=jax.ShapeDtypeStruct((32, 1, N_H), jnp.float32),
        mesh=_mesh(),
        scratch_types=[
            pltpu.VMEM((EW,), jnp.int32),
            pltpu.VMEM((1, N_H), jnp.float32),
            pltpu.SemaphoreType.DMA,
        ],
        compiler_params=dataclasses.replace(pltpu.CompilerParams(),
                                            needs_layout_passes=False),
    )
    def k(c_hbm, out_hbm, colb, hv, isem):
        cid = lax.axis_index("c")
        sid = lax.axis_index("s")
        wid = sid * 2 + cid
        cp = pltpu.async_copy(c_hbm.at[pl.ds(wid * EW, EW)], colb, isem)

        z16 = jnp.zeros((16,), jnp.float32)

        @pl.loop(0, N_H // 16)
        def _(i):
            hv[0, pl.ds(i * 16, 16)] = z16

        cp.wait()
        zi = jnp.zeros((16,), jnp.int32)
        o16 = jnp.ones((16,), jnp.float32)

        @pl.loop(0, EW // 16)
        def _(i):
            idx = colb[pl.ds(i * 16, 16)]
            plsc.addupdate_scatter(hv, [zi, idx], o16)

        pltpu.sync_copy(hv, out_hbm.at[wid])

    return k(colsf)


def _dis(hist32):
    bh = 2560

    def body(h_ref, o_ref):
        s = jnp.sum(h_ref[...], axis=0, keepdims=True) + 1.0
        o_ref[...] = lax.rsqrt(s).reshape(bh, 1)

    return pl.pallas_call(
        body,
        grid=(N_H // bh,),
        in_specs=[pl.BlockSpec((32, bh), lambda i: (0, i))],
        out_specs=pl.BlockSpec((bh, 1), lambda i: (i, 0)),
        out_shape=jax.ShapeDtypeStruct((N_H, 1), jnp.float32),
    )(hist32)


ZR = 48


def _sc_agg(y, rows2, cols2):
    @functools.partial(
        pl.kernel,
        out_type=jax.ShapeDtypeStruct((2 * N, D), jnp.float32),
        mesh=_mesh(),
        scratch_types=[
            pltpu.VMEM((G, K), jnp.int32),
            pltpu.VMEM((G, K), jnp.int32),
            pltpu.VMEM((W_GATHER, K, D), jnp.float32),
            pltpu.VMEM((ZR, D), jnp.float32),
            pltpu.VMEM_SHARED((N_ACC, D), jnp.float32),
            pltpu.SemaphoreType.DMA,
            pltpu.SemaphoreType.DMA,
            pltpu.SemaphoreType.DMA,
            pltpu.SemaphoreType.DMA,
            pltpu.SemaphoreType.DMA,
        ],
    )
    def k(y_hbm, r_hbm, c_hbm, out_hbm, rowb0, colb0, gbuf,
          zbuf, acc, gsem0, gsem1, ssem0, ssem1, zsem):
        cid = lax.axis_index("c")
        sid = lax.axis_index("s")
        wid = sid * 2 + cid
        rbase = sid * RPS

        z16 = jnp.zeros((16,), jnp.float32)

        @pl.loop(0, ZR)
        def _(r):
            @pl.loop(0, D // 16)
            def _(c):
                zbuf[r, pl.ds(c * 16, 16)] = z16

        zcp = [pltpu.async_copy(zbuf, acc.at[pl.ds(rbase + i * ZR, ZR)], zsem)
               for i in range(RPS // ZR)]

        @pl.when(sid == 0)
        def _():
            pltpu.async_copy(zbuf.at[pl.ds(0, RTAIL)],
                             acc.at[pl.ds(16 * RPS, RTAIL)], zsem).wait()

        cbase = wid * WCH
        gsems = (gsem0, gsem1)
        ssems = (ssem0, ssem1)

        def process(rowb, colb):
            gcp = [None] * G
            scp = [None] * G
            gcp[0] = pltpu.async_copy(y_hbm.at[rowb.at[0]], gbuf.at[0], gsems[0])
            gcp[1] = pltpu.async_copy(y_hbm.at[rowb.at[1]], gbuf.at[1], gsems[1])
            for j in range(G):
                b = j % 2
                if 1 <= j < G - 1:
                    scp[j - 1].wait()
                    gcp[j + 1] = pltpu.async_copy(y_hbm.at[rowb.at[j + 1]],
                                                  gbuf.at[1 - b], gsems[1 - b])
                gcp[j].wait()
                scp[j] = pltpu.async_copy(gbuf.at[b], acc.at[colb.at[j]],
                                          ssems[b], add=True)
            scp[G - 2].wait()
            scp[G - 1].wait()

        for cp in zcp:
            cp.wait()
        plsc.subcore_barrier()

        @pl.loop(0, WG)
        def _(g):
            ch = cbase + g * G
            pltpu.sync_copy(r_hbm.at[pl.ds(ch, G)], rowb0)
            pltpu.sync_copy(c_hbm.at[pl.ds(ch, G)], colb0)
            process(rowb0, colb0)

        plsc.subcore_barrier()
        pltpu.sync_copy(acc.at[pl.ds(rbase, RPS)],
                        out_hbm.at[pl.ds(cid * N + rbase, RPS)])

        @pl.when(sid == 0)
        def _():
            pltpu.sync_copy(acc.at[pl.ds(16 * RPS, RTAIL)],
                            out_hbm.at[pl.ds(cid * N + 16 * RPS, RTAIL)])

    return k(y, rows2, cols2)


def _mm_first(x, w, dis):
    def body(x_ref, w_ref, d_ref, o_ref):
        o_ref[...] = jnp.dot(x_ref[...], w_ref[...],
                             precision=lax.Precision.HIGHEST) * d_ref[...]

    return pl.pallas_call(
        body,
        grid=(NB,),
        in_specs=[
            pl.BlockSpec((BN, D), lambda i: (i, 0)),
            pl.BlockSpec((D, D), lambda i: (0, 0)),
            pl.BlockSpec((BN, 1), lambda i: (i, 0)),
        ],
        out_specs=pl.BlockSpec((BN, D), lambda i: (i, 0)),
        out_shape=jax.ShapeDtypeStruct((N, D), jnp.float32),
    )(x, w, dis)


def _mm_mid(a, y, dis, b1, w2):
    def body(p0_ref, p1_ref, y_ref, d_ref, b_ref, w_ref, o_ref):
        dis = d_ref[...]
        full = p0_ref[...] + p1_ref[...] + y_ref[...]
        hid = jnp.maximum(full * dis + b_ref[...], 0.0)
        o_ref[...] = jnp.dot(hid, w_ref[...],
                             precision=lax.Precision.HIGHEST) * dis

    return pl.pallas_call(
        body,
        grid=(NB,),
        in_specs=[
            pl.BlockSpec((BN, D), lambda i: (i, 0)),
            pl.BlockSpec((BN, D), lambda i: (i + NB, 0)),
            pl.BlockSpec((BN, D), lambda i: (i, 0)),
            pl.BlockSpec((BN, 1), lambda i: (i, 0)),
            pl.BlockSpec((1, D), lambda i: (0, 0)),
            pl.BlockSpec((D, D), lambda i: (0, 0)),
        ],
        out_specs=pl.BlockSpec((BN, D), lambda i: (i, 0)),
        out_shape=jax.ShapeDtypeStruct((N, D), jnp.float32),
    )(a, a, y, dis, b1, w2)


def _mm_final(a, y, dis, b2, wh, bh):
    def body(p0_ref, p1_ref, y_ref, d_ref, b_ref, w_ref, bh_ref, o_ref):
        dis = d_ref[...]
        full = p0_ref[...] + p1_ref[...] + y_ref[...]
        z = full * dis + b_ref[...]
        o_ref[...] = jnp.dot(z, w_ref[...],
                             precision=lax.Precision.HIGHEST) + bh_ref[...]

    return pl.pallas_call(
        body,
        grid=(NB,),
        in_specs=[
            pl.BlockSpec((BN, D), lambda i: (i, 0)),
            pl.BlockSpec((BN, D), lambda i: (i + NB, 0)),
            pl.BlockSpec((BN, D), lambda i: (i, 0)),
            pl.BlockSpec((BN, 1), lambda i: (i, 0)),
            pl.BlockSpec((1, D), lambda i: (0, 0)),
            pl.BlockSpec((D, D), lambda i: (0, 0)),
            pl.BlockSpec((1, D), lambda i: (0, 0)),
        ],
        out_specs=pl.BlockSpec((BN, D), lambda i: (i, 0)),
        out_shape=jax.ShapeDtypeStruct((N, D), jnp.float32),
    )(a, a, y, dis, b2, wh, bh)


def kernel(x, edge_index, W1, b1, W2, b2, Wh, bh):
    row = edge_index[0]
    col = edge_index[1]
    pad = E_PAD - E
    pad_iota = jnp.arange(pad, dtype=jnp.int32)
    rows2 = jnp.concatenate([row, pad_iota % N]).reshape(NCHT, K)
    cols_p = jnp.concatenate([col, N + pad_iota % (N_ACC - N)])
    cols2 = cols_p.reshape(NCHT, K)
    b1r = b1.reshape(1, D)
    b2r = b2.reshape(1, D)
    bhr = bh.reshape(1, D)

    hist32 = _sc_hist(cols_p).reshape(32, N_H)
    dis = _dis(hist32)
    y1 = _mm_first(x, W1, dis)
    a1 = _sc_agg(y1, rows2, cols2)
    y2 = _mm_mid(a1, y1, dis, b1r, W2)
    a2 = _sc_agg(y2, rows2, cols2)
    return _mm_final(a2, y2, dis, b2r, Wh, bhr)


# speedup vs baseline: 1.0623x; 1.0199x over previous
import dataclasses
import functools

import jax
import jax.numpy as jnp
from jax import lax
from jax.experimental import pallas as pl
from jax.experimental.pallas import tpu as pltpu
from jax.experimental.pallas import tpu_sc as plsc

N = 10000
D = 128
E = 320000
K = 128
G = 16
W_GATHER = 2
E_PAD = 327680
NCHT = E_PAD // K
WCH = NCHT // 32
WG = WCH // G
RPS = 624
RTAIL = N - 16 * RPS
N_ACC = 10240
BN = 2000
NB = N // BN


def _mesh():
    return plsc.VectorSubcoreMesh(core_axis_name="c", subcore_axis_name="s")


EW = E_PAD // 32
N_H = N_ACC


def _sc_hist(colsf):
    @functools.partial(
        pl.kernel,
        out_type=jax.ShapeDtypeStruct((32, 1, N_H), jnp.float32),
        mesh=_mesh(),
        scratch_types=[
            pltpu.VMEM((EW,), jnp.int32),
            pltpu.VMEM((1, N_H), jnp.float32),
            pltpu.SemaphoreType.DMA,
        ],
        compiler_params=dataclasses.replace(pltpu.CompilerParams(),
                                            needs_layout_passes=False),
    )
    def k(c_hbm, out_hbm, colb, hv, isem):
        cid = lax.axis_index("c")
        sid = lax.axis_index("s")
        wid = sid * 2 + cid
        cp = pltpu.async_copy(c_hbm.at[pl.ds(wid * EW, EW)], colb, isem)

        z16 = jnp.zeros((16,), jnp.float32)

        @pl.loop(0, N_H // 16)
        def _(i):
            hv[0, pl.ds(i * 16, 16)] = z16

        cp.wait()
        zi = jnp.zeros((16,), jnp.int32)
        o16 = jnp.ones((16,), jnp.float32)

        @pl.loop(0, EW // 16)
        def _(i):
            idx = colb[pl.ds(i * 16, 16)]
            plsc.addupdate_scatter(hv, [zi, idx], o16)

        pltpu.sync_copy(hv, out_hbm.at[wid])

    return k(colsf)


def _dis(hist32):
    bh = 2560

    def body(h_ref, o_ref):
        s = jnp.sum(h_ref[...], axis=0, keepdims=True) + 1.0
        o_ref[...] = lax.rsqrt(s).reshape(bh, 1)

    return pl.pallas_call(
        body,
        grid=(N_H // bh,),
        in_specs=[pl.BlockSpec((32, bh), lambda i: (0, i))],
        out_specs=pl.BlockSpec((bh, 1), lambda i: (i, 0)),
        out_shape=jax.ShapeDtypeStruct((N_H, 1), jnp.float32),
    )(hist32)


ZR = 48


def _sc_agg(y, rows2, cols2):
    @functools.partial(
        pl.kernel,
        out_type=jax.ShapeDtypeStruct((2 * N, D), jnp.float32),
        mesh=_mesh(),
        scratch_types=[
            pltpu.VMEM((G, K), jnp.int32),
            pltpu.VMEM((G, K), jnp.int32),
            pltpu.VMEM((W_GATHER, K, D), jnp.float32),
            pltpu.VMEM((ZR, D), jnp.float32),
            pltpu.VMEM_SHARED((N_ACC, D), jnp.float32),
            pltpu.SemaphoreType.DMA,
            pltpu.SemaphoreType.DMA,
            pltpu.SemaphoreType.DMA,
            pltpu.SemaphoreType.DMA,
            pltpu.SemaphoreType.DMA,
        ],
    )
    def k(y_hbm, r_hbm, c_hbm, out_hbm, rowb0, colb0, gbuf,
          zbuf, acc, gsem0, gsem1, ssem0, ssem1, zsem):
        cid = lax.axis_index("c")
        sid = lax.axis_index("s")
        wid = sid * 2 + cid
        rbase = sid * RPS

        z16 = jnp.zeros((16,), jnp.float32)

        @pl.loop(0, ZR)
        def _(r):
            @pl.loop(0, D // 16)
            def _(c):
                zbuf[r, pl.ds(c * 16, 16)] = z16

        zcp = [pltpu.async_copy(zbuf, acc.at[pl.ds(rbase + i * ZR, ZR)], zsem)
               for i in range(RPS // ZR)]

        @pl.when(sid == 0)
        def _():
            pltpu.async_copy(zbuf.at[pl.ds(0, RTAIL)],
                             acc.at[pl.ds(16 * RPS, RTAIL)], zsem).wait()

        cbase = wid * WCH
        gsems = (gsem0, gsem1)
        ssems = (ssem0, ssem1)

        def process(rowb, colb):
            gcp = [None] * G
            scp = [None] * G
            gcp[0] = pltpu.async_copy(y_hbm.at[rowb.at[0]], gbuf.at[0], gsems[0])
            gcp[1] = pltpu.async_copy(y_hbm.at[rowb.at[1]], gbuf.at[1], gsems[1])
            for j in range(G):
                b = j % 2
                if 1 <= j < G - 1:
                    scp[j - 1].wait()
                    gcp[j + 1] = pltpu.async_copy(y_hbm.at[rowb.at[j + 1]],
                                                  gbuf.at[1 - b], gsems[1 - b])
                gcp[j].wait()
                scp[j] = pltpu.async_copy(gbuf.at[b], acc.at[colb.at[j]],
                                          ssems[b], add=True)
            scp[G - 2].wait()
            scp[G - 1].wait()

        for cp in zcp:
            cp.wait()
        plsc.subcore_barrier()

        @pl.loop(0, WG)
        def _(g):
            ch = cbase + g * G
            pltpu.sync_copy(r_hbm.at[pl.ds(ch, G)], rowb0)
            pltpu.sync_copy(c_hbm.at[pl.ds(ch, G)], colb0)
            process(rowb0, colb0)

        plsc.subcore_barrier()
        pltpu.sync_copy(acc.at[pl.ds(rbase, RPS)],
                        out_hbm.at[pl.ds(cid * N + rbase, RPS)])

        @pl.when(sid == 0)
        def _():
            pltpu.sync_copy(acc.at[pl.ds(16 * RPS, RTAIL)],
                            out_hbm.at[pl.ds(cid * N + 16 * RPS, RTAIL)])

    return k(y, rows2, cols2)


def _mm_first(x, w, dis):
    def body(x_ref, w_ref, d_ref, o_ref):
        o_ref[...] = jnp.dot(x_ref[...], w_ref[...]) * d_ref[...]

    return pl.pallas_call(
        body,
        grid=(NB,),
        in_specs=[
            pl.BlockSpec((BN, D), lambda i: (i, 0)),
            pl.BlockSpec((D, D), lambda i: (0, 0)),
            pl.BlockSpec((BN, 1), lambda i: (i, 0)),
        ],
        out_specs=pl.BlockSpec((BN, D), lambda i: (i, 0)),
        out_shape=jax.ShapeDtypeStruct((N, D), jnp.float32),
    )(x, w, dis)


def _mm_mid(a, y, dis, b1, w2):
    def body(p0_ref, p1_ref, y_ref, d_ref, b_ref, w_ref, o_ref):
        dis = d_ref[...]
        full = p0_ref[...] + p1_ref[...] + y_ref[...]
        hid = jnp.maximum(full * dis + b_ref[...], 0.0)
        o_ref[...] = jnp.dot(hid, w_ref[...]) * dis

    return pl.pallas_call(
        body,
        grid=(NB,),
        in_specs=[
            pl.BlockSpec((BN, D), lambda i: (i, 0)),
            pl.BlockSpec((BN, D), lambda i: (i + NB, 0)),
            pl.BlockSpec((BN, D), lambda i: (i, 0)),
            pl.BlockSpec((BN, 1), lambda i: (i, 0)),
            pl.BlockSpec((1, D), lambda i: (0, 0)),
            pl.BlockSpec((D, D), lambda i: (0, 0)),
        ],
        out_specs=pl.BlockSpec((BN, D), lambda i: (i, 0)),
        out_shape=jax.ShapeDtypeStruct((N, D), jnp.float32),
    )(a, a, y, dis, b1, w2)


def _mm_final(a, y, dis, b2, wh, bh):
    def body(p0_ref, p1_ref, y_ref, d_ref, b_ref, w_ref, bh_ref, o_ref):
        dis = d_ref[...]
        full = p0_ref[...] + p1_ref[...] + y_ref[...]
        z = full * dis + b_ref[...]
        o_ref[...] = jnp.dot(z, w_ref[...]) + bh_ref[...]

    return pl.pallas_call(
        body,
        grid=(NB,),
        in_specs=[
            pl.BlockSpec((BN, D), lambda i: (i, 0)),
            pl.BlockSpec((BN, D), lambda i: (i + NB, 0)),
            pl.BlockSpec((BN, D), lambda i: (i, 0)),
            pl.BlockSpec((BN, 1), lambda i: (i, 0)),
            pl.BlockSpec((1, D), lambda i: (0, 0)),
            pl.BlockSpec((D, D), lambda i: (0, 0)),
            pl.BlockSpec((1, D), lambda i: (0, 0)),
        ],
        out_specs=pl.BlockSpec((BN, D), lambda i: (i, 0)),
        out_shape=jax.ShapeDtypeStruct((N, D), jnp.float32),
    )(a, a, y, dis, b2, wh, bh)


def kernel(x, edge_index, W1, b1, W2, b2, Wh, bh):
    row = edge_index[0]
    col = edge_index[1]
    pad = E_PAD - E
    pad_iota = jnp.arange(pad, dtype=jnp.int32)
    rows2 = jnp.concatenate([row, pad_iota % N]).reshape(NCHT, K)
    cols_p = jnp.concatenate([col, N + pad_iota % (N_ACC - N)])
    cols2 = cols_p.reshape(NCHT, K)
    b1r = b1.reshape(1, D)
    b2r = b2.reshape(1, D)
    bhr = bh.reshape(1, D)

    hist32 = _sc_hist(cols_p).reshape(32, N_H)
    dis = _dis(hist32)
    y1 = _mm_first(x, W1, dis)
    a1 = _sc_agg(y1, rows2, cols2)
    y2 = _mm_mid(a1, y1, dis, b1r, W2)
    a2 = _sc_agg(y2, rows2, cols2)
    return _mm_final(a2, y2, dis, b2r, Wh, bhr)
